# Initial kernel scaffold; baseline (speedup 1.0000x reference)
import jax
import jax.numpy as jnp
from jax.experimental import pallas as pl


def kernel(x, edge_index_onset, edge_index_consecutive, edge_index_sustain, edge_index_silence, edge_index_voice, W1_onset, b1_onset, W2_onset, b2_onset, W1_consecutive, b1_consecutive, W2_consecutive, b2_consecutive, W1_sustain, b1_sustain, W2_sustain, b2_sustain, W1_silence, b1_silence, W2_silence, b2_silence, W1_voice, b1_voice, W2_voice, b2_voice):
    raise NotImplementedError("write your pallas kernel here")


# speedup vs baseline: 19.1125x; 19.1125x over previous
import functools

import jax
import jax.numpy as jnp
from jax import lax
from jax.experimental import pallas as pl
from jax.experimental.pallas import tpu as pltpu
from jax.experimental.pallas import tpu_sc as plsc

N = 50000
E = 800000
NE = 5
D = 24
HID = 128
OUT = 64

NP = 50176
NC = 2
NS = 16
ROWS_PER_SUB = NP // NS

B = 125
CH = E // B
CPW = CH // (NC * NS)
G = 8
GPW = CPW // G

CN = 1024
GRID = NP // CN


def _agg_body(xp, er0, er1, er2, er3, er4, za, out, idx_v, rows_v, agg_sh, sems):
    cid = lax.axis_index("c")
    sid = lax.axis_index("s")
    w = sid * NC + cid
    row0 = sid * ROWS_PER_SUB
    ers = (er0, er1, er2, er3, er4)
    for e in range(NE):
        pltpu.sync_copy(za.at[pl.ds(row0, ROWS_PER_SUB), :],
                        agg_sh.at[pl.ds(row0, ROWS_PER_SUB), :])
        plsc.subcore_barrier()

        def _group(g, carry, e=e):
            base = w * CPW + g * G
            pltpu.sync_copy(ers[e].at[:, pl.ds(base, G), :], idx_v)
            descs = []
            for j in range(G):
                descs.append(pltpu.async_copy(
                    xp.at[idx_v.at[0, j]], rows_v.at[j], sems.at[j]))
            for j in range(G):
                descs[j].wait()
                pltpu.sync_copy(rows_v.at[j], agg_sh.at[idx_v.at[1, j]],
                                add=True)
            return carry

        lax.fori_loop(0, GPW, _group, 0)
        plsc.subcore_barrier()
        pltpu.sync_copy(agg_sh.at[pl.ds(row0, ROWS_PER_SUB), :],
                        out.at[cid, e, pl.ds(row0, ROWS_PER_SUB), :])
        plsc.subcore_barrier()


def _u_body(c0, c1, c2, c3, c4, er0, er1, er2, er3, er4, zu, out,
            idx_v, cv, u_sh0, u_sh1, u_sh2, u_sh3, u_sh4, sems):
    cid = lax.axis_index("c")
    sid = lax.axis_index("s")
    w = sid * NC + cid
    row0 = sid * ROWS_PER_SUB
    cs = (c0, c1, c2, c3, c4)
    ers = (er0, er1, er2, er3, er4)
    u_shs = (u_sh0, u_sh1, u_sh2, u_sh3, u_sh4)
    for e in range(NE):
        pltpu.sync_copy(zu.at[pl.ds(row0, ROWS_PER_SUB)],
                        u_shs[e].at[pl.ds(row0, ROWS_PER_SUB)])
    plsc.subcore_barrier()
    for e in range(NE):
        def _group(g, carry, e=e):
            base = w * CPW + g * G
            pltpu.sync_copy(ers[e].at[:, pl.ds(base, G), :], idx_v)
            descs = []
            for j in range(G):
                descs.append(pltpu.async_copy(
                    cs[e].at[idx_v.at[1, j]], cv.at[j], sems.at[j]))
            for j in range(G):
                descs[j].wait()
                pltpu.sync_copy(cv.at[j], u_shs[e].at[idx_v.at[0, j]],
                                add=True)
            return carry

        lax.fori_loop(0, GPW, _group, 0)
    plsc.subcore_barrier()
    for e in range(NE):
        pltpu.sync_copy(u_shs[e].at[pl.ds(row0, ROWS_PER_SUB)],
                        out.at[cid, e, pl.ds(row0, ROWS_PER_SUB)])


def _tc_body(aggp, xp, c, v8, w1, b1, w2, b2, out, sacc):
    i = pl.program_id(0)

    @pl.when(i == 0)
    def _():
        sacc[...] = jnp.zeros_like(sacc)

    agg = aggp[0] + aggp[1]
    xb = xp[...]
    cb = c[...]
    h1 = b1[...]
    for e in range(NE):
        ze = (agg[e] + xb) * cb[e][:, None]
        h1 = h1 + jnp.dot(ze, w1[e * D:(e + 1) * D, :],
                          preferred_element_type=jnp.float32)
    h1 = jnp.maximum(h1, 0.0)
    sacc[...] += jnp.dot(v8[...], h1, preferred_element_type=jnp.float32)

    @pl.when(i == GRID - 1)
    def _():
        s = sacc[...]
        acc = jnp.zeros((1, OUT), dtype=jnp.float32)
        for e in range(NE):
            acc = acc + jnp.dot(s[e][None, :], w2[e * HID:(e + 1) * HID, :],
                                preferred_element_type=jnp.float32)
        out[...] = acc * (1.0 / N) + b2[...]


_mesh = plsc.VectorSubcoreMesh(core_axis_name="c", subcore_axis_name="s")

_agg_kernel = pl.kernel(
    _agg_body,
    out_type=jax.ShapeDtypeStruct((NC, NE, NP, D), jnp.float32),
    mesh=_mesh,
    scratch_types=[
        pltpu.VMEM((2, G, B), jnp.int32),
        pltpu.VMEM((G, B, D), jnp.float32),
        pltpu.VMEM_SHARED((NP, D), jnp.float32),
        pltpu.SemaphoreType.DMA((G,)),
    ],
    compiler_params=pltpu.CompilerParams(use_tc_tiling_on_sc=False),
)

_u_kernel = pl.kernel(
    _u_body,
    out_type=jax.ShapeDtypeStruct((NC, NE, NP), jnp.float32),
    mesh=_mesh,
    scratch_types=(
        [pltpu.VMEM((2, G, B), jnp.int32), pltpu.VMEM((G, B), jnp.float32)]
        + [pltpu.VMEM_SHARED((NP,), jnp.float32) for _ in range(NE)]
        + [pltpu.SemaphoreType.DMA((G,))]
    ),
    compiler_params=pltpu.CompilerParams(use_tc_tiling_on_sc=False),
)

_tc_kernel = pl.pallas_call(
    _tc_body,
    grid=(GRID,),
    in_specs=[
        pl.BlockSpec((NC, NE, CN, D), lambda i: (0, 0, i, 0)),
        pl.BlockSpec((CN, D), lambda i: (i, 0)),
        pl.BlockSpec((NE, CN), lambda i: (0, i)),
        pl.BlockSpec((8, CN), lambda i: (0, i)),
        pl.BlockSpec((NE * D, HID), lambda i: (0, 0)),
        pl.BlockSpec((1, HID), lambda i: (0, 0)),
        pl.BlockSpec((NE * HID, OUT), lambda i: (0, 0)),
        pl.BlockSpec((1, OUT), lambda i: (0, 0)),
    ],
    out_specs=pl.BlockSpec((1, OUT), lambda i: (0, 0)),
    out_shape=jax.ShapeDtypeStruct((1, OUT), jnp.float32),
    scratch_shapes=[pltpu.VMEM((8, HID), jnp.float32)],
)


def kernel(x, edge_index_onset, edge_index_consecutive, edge_index_sustain,
           edge_index_silence, edge_index_voice, W1_onset, b1_onset,
           W2_onset, b2_onset, W1_consecutive, b1_consecutive,
           W2_consecutive, b2_consecutive, W1_sustain, b1_sustain,
           W2_sustain, b2_sustain, W1_silence, b1_silence, W2_silence,
           b2_silence, W1_voice, b1_voice, W2_voice, b2_voice):
    edges = (edge_index_onset, edge_index_consecutive, edge_index_sustain,
             edge_index_silence, edge_index_voice)
    W1s = (W1_onset, W1_consecutive, W1_sustain, W1_silence, W1_voice)
    b1s = (b1_onset, b1_consecutive, b1_sustain, b1_silence, b1_voice)
    W2s = (W2_onset, W2_consecutive, W2_sustain, W2_silence, W2_voice)
    b2s = (b2_onset, b2_consecutive, b2_sustain, b2_silence, b2_voice)

    xp = jnp.pad(jnp.concatenate(
        [x, jnp.ones((N, 1), jnp.float32)], axis=1),
        ((0, NP - N), (0, 0)))
    ers = [e.reshape(2, CH, B) for e in edges]
    za = jnp.zeros((NP, D), jnp.float32)
    zu = jnp.zeros((NP,), jnp.float32)

    agg_part = _agg_kernel(xp, *ers, za)

    deg = agg_part[0, :, :, D - 1] + agg_part[1, :, :, D - 1]
    c = 1.0 / (deg + 1.0)

    u_part = _u_kernel(*[c[e] for e in range(NE)], *ers, zu)

    mask = (jnp.arange(NP) < N).astype(jnp.float32)
    v = (u_part[0] + u_part[1] + c) * mask[None, :]
    v8 = jnp.concatenate([v, jnp.zeros((8 - NE, NP), jnp.float32)], axis=0)

    w1cat = jnp.concatenate(
        [jnp.concatenate([w, jnp.zeros((1, HID), jnp.float32)], axis=0)
         for w in W1s], axis=0)
    b1sum = functools.reduce(jnp.add, b1s)[None, :]
    w2cat = jnp.concatenate(W2s, axis=0)
    b2sum = functools.reduce(jnp.add, b2s)[None, :]

    return _tc_kernel(agg_part, xp, c, v8, w1cat, b1sum, w2cat, b2sum)


# speedup vs baseline: 25.8102x; 1.3504x over previous
import functools

import jax
import jax.numpy as jnp
from jax import lax
from jax.experimental import pallas as pl
from jax.experimental.pallas import tpu as pltpu
from jax.experimental.pallas import tpu_sc as plsc

N = 50000
E = 800000
NE = 5
D = 24
HID = 128
OUT = 64

NP = 50176
NC = 2
NS = 16
ROWS_PER_SUB = NP // NS

B = 125
CH = E // B
CPW = CH // (NC * NS)
G = 10
GPW = CPW // G

CN = 1024
GRID = NP // CN


def _agg_body(xp, er0, er1, er2, er3, er4, za, out,
              idx_v, rows_v, agg_sh, gsems, ssems):
    cid = lax.axis_index("c")
    sid = lax.axis_index("s")
    w = sid * NC + cid
    row0 = sid * ROWS_PER_SUB
    ers = (er0, er1, er2, er3, er4)
    for e in range(NE):
        @pl.when(cid == 0)
        def _():
            pltpu.sync_copy(xp.at[pl.ds(row0, ROWS_PER_SUB), :],
                            agg_sh.at[pl.ds(row0, ROWS_PER_SUB), :])

        @pl.when(cid != 0)
        def _():
            pltpu.sync_copy(za.at[pl.ds(row0, ROWS_PER_SUB), :],
                            agg_sh.at[pl.ds(row0, ROWS_PER_SUB), :])

        plsc.subcore_barrier()

        def _group(g, carry, e=e):
            base = w * CPW + g * G
            pltpu.sync_copy(ers[e].at[:, pl.ds(base, G), :], idx_v)
            gd, sd = [], []
            for j in range(G):
                gd.append(pltpu.async_copy(
                    xp.at[idx_v.at[0, j]], rows_v.at[j], gsems.at[j]))
            for j in range(G):
                gd[j].wait()
                sd.append(pltpu.async_copy(
                    rows_v.at[j], agg_sh.at[idx_v.at[1, j]], ssems.at[j],
                    add=True))
            for j in range(G):
                sd[j].wait()
            return carry

        lax.fori_loop(0, GPW, _group, 0)
        plsc.subcore_barrier()
        pltpu.sync_copy(agg_sh.at[pl.ds(row0, ROWS_PER_SUB), :],
                        out.at[cid, pl.ds(row0, ROWS_PER_SUB),
                               pl.ds(e * D, D)])
        plsc.subcore_barrier()


def _u_body(c, er0, er1, er2, er3, er4, zu, out,
            idx_v, cv, c_sh0, c_sh1, c_sh2, c_sh3, c_sh4,
            u_sh0, u_sh1, u_sh2, u_sh3, u_sh4, gsems, ssems):
    cid = lax.axis_index("c")
    sid = lax.axis_index("s")
    w = sid * NC + cid
    row0 = sid * ROWS_PER_SUB
    ers = (er0, er1, er2, er3, er4)
    c_shs = (c_sh0, c_sh1, c_sh2, c_sh3, c_sh4)
    u_shs = (u_sh0, u_sh1, u_sh2, u_sh3, u_sh4)
    for e in range(NE):
        pltpu.sync_copy(zu.at[pl.ds(row0, ROWS_PER_SUB)],
                        u_shs[e].at[pl.ds(row0, ROWS_PER_SUB)])
        pltpu.sync_copy(c.at[e, pl.ds(row0, ROWS_PER_SUB)],
                        c_shs[e].at[pl.ds(row0, ROWS_PER_SUB)])
    plsc.subcore_barrier()
    for e in range(NE):
        def _group(g, carry, e=e):
            base = w * CPW + g * G
            pltpu.sync_copy(ers[e].at[:, pl.ds(base, G), :], idx_v)
            gd, sd = [], []
            for j in range(G):
                gd.append(pltpu.async_copy(
                    c_shs[e].at[idx_v.at[1, j]], cv.at[j], gsems.at[j]))
            for j in range(G):
                gd[j].wait()
                sd.append(pltpu.async_copy(
                    cv.at[j], u_shs[e].at[idx_v.at[0, j]], ssems.at[j],
                    add=True))
            for j in range(G):
                sd[j].wait()
            return carry

        lax.fori_loop(0, GPW, _group, 0)
    plsc.subcore_barrier()
    for e in range(NE):
        pltpu.sync_copy(u_shs[e].at[pl.ds(row0, ROWS_PER_SUB)],
                        out.at[cid, e, pl.ds(row0, ROWS_PER_SUB)])


def _tc_body(aggp, ct8, sel, v8, w1, b1, w2, b2, out, sacc):
    i = pl.program_id(0)

    @pl.when(i == 0)
    def _():
        sacc[...] = jnp.zeros_like(sacc)

    agg = aggp[0] + aggp[1]
    ce = jnp.dot(ct8[...], sel[...],
                 preferred_element_type=jnp.float32)
    cols = lax.broadcasted_iota(jnp.int32, (CN, HID), 1)
    z = jnp.where(cols < NE * D, agg * ce, 0.0)
    h1 = jnp.maximum(jnp.dot(z, w1[...],
                             preferred_element_type=jnp.float32) + b1[...],
                     0.0)
    sacc[...] += jnp.dot(v8[...], h1, preferred_element_type=jnp.float32)

    @pl.when(i == GRID - 1)
    def _():
        s = sacc[...]
        acc = jnp.zeros((1, OUT), dtype=jnp.float32)
        for e in range(NE):
            acc = acc + jnp.dot(s[e][None, :], w2[e * HID:(e + 1) * HID, :],
                                preferred_element_type=jnp.float32)
        out[...] = acc * (1.0 / N) + b2[...]


_mesh = plsc.VectorSubcoreMesh(core_axis_name="c", subcore_axis_name="s")

_agg_kernel = pl.kernel(
    _agg_body,
    out_type=jax.ShapeDtypeStruct((NC, NP, HID), jnp.float32),
    mesh=_mesh,
    scratch_types=[
        pltpu.VMEM((2, G, B), jnp.int32),
        pltpu.VMEM((G, B, D), jnp.float32),
        pltpu.VMEM_SHARED((NP, D), jnp.float32),
        pltpu.SemaphoreType.DMA((G,)),
        pltpu.SemaphoreType.DMA((G,)),
    ],
    compiler_params=pltpu.CompilerParams(use_tc_tiling_on_sc=False),
)

_u_kernel = pl.kernel(
    _u_body,
    out_type=jax.ShapeDtypeStruct((NC, NE, NP), jnp.float32),
    mesh=_mesh,
    scratch_types=(
        [pltpu.VMEM((2, G, B), jnp.int32), pltpu.VMEM((G, B), jnp.float32)]
        + [pltpu.VMEM_SHARED((NP,), jnp.float32) for _ in range(2 * NE)]
        + [pltpu.SemaphoreType.DMA((G,)) for _ in range(2)]
    ),
    compiler_params=pltpu.CompilerParams(use_tc_tiling_on_sc=False),
)

_tc_kernel = pl.pallas_call(
    _tc_body,
    grid=(GRID,),
    in_specs=[
        pl.BlockSpec((NC, CN, HID), lambda i: (0, i, 0)),
        pl.BlockSpec((CN, 8), lambda i: (i, 0)),
        pl.BlockSpec((8, HID), lambda i: (0, 0)),
        pl.BlockSpec((8, CN), lambda i: (0, i)),
        pl.BlockSpec((HID, HID), lambda i: (0, 0)),
        pl.BlockSpec((1, HID), lambda i: (0, 0)),
        pl.BlockSpec((NE * HID, OUT), lambda i: (0, 0)),
        pl.BlockSpec((1, OUT), lambda i: (0, 0)),
    ],
    out_specs=pl.BlockSpec((1, OUT), lambda i: (0, 0)),
    out_shape=jax.ShapeDtypeStruct((1, OUT), jnp.float32),
    scratch_shapes=[pltpu.VMEM((8, HID), jnp.float32)],
)


def kernel(x, edge_index_onset, edge_index_consecutive, edge_index_sustain,
           edge_index_silence, edge_index_voice, W1_onset, b1_onset,
           W2_onset, b2_onset, W1_consecutive, b1_consecutive,
           W2_consecutive, b2_consecutive, W1_sustain, b1_sustain,
           W2_sustain, b2_sustain, W1_silence, b1_silence, W2_silence,
           b2_silence, W1_voice, b1_voice, W2_voice, b2_voice):
    edges = (edge_index_onset, edge_index_consecutive, edge_index_sustain,
             edge_index_silence, edge_index_voice)
    W1s = (W1_onset, W1_consecutive, W1_sustain, W1_silence, W1_voice)
    b1s = (b1_onset, b1_consecutive, b1_sustain, b1_silence, b1_voice)
    W2s = (W2_onset, W2_consecutive, W2_sustain, W2_silence, W2_voice)
    b2s = (b2_onset, b2_consecutive, b2_sustain, b2_silence, b2_voice)

    xp = jnp.pad(jnp.concatenate(
        [x, jnp.ones((N, 1), jnp.float32)], axis=1),
        ((0, NP - N), (0, 0)))
    ers = [e.reshape(2, CH, B) for e in edges]
    za = jnp.zeros((NP, D), jnp.float32)
    zu = jnp.zeros((NP,), jnp.float32)

    agg_part = _agg_kernel(xp, *ers, za)

    degp1 = agg_part[0, :, D - 1::D] + agg_part[1, :, D - 1::D]
    ct = 1.0 / jnp.maximum(degp1, 1.0)
    c = ct.T
    ct8 = jnp.concatenate(
        [ct, jnp.zeros((NP, 8 - NE), jnp.float32)], axis=1)

    u_part = _u_kernel(c, *ers, zu)

    mask = (jnp.arange(NP) < N).astype(jnp.float32)
    v = (u_part[0] + u_part[1] + c) * mask[None, :]
    v8 = jnp.concatenate([v, jnp.zeros((8 - NE, NP), jnp.float32)], axis=0)

    sel = jnp.zeros((8, HID), jnp.float32)
    for e in range(NE):
        sel = sel.at[e, e * D:(e + 1) * D].set(1.0)
    w1cat = jnp.concatenate(
        [jnp.concatenate([w, jnp.zeros((1, HID), jnp.float32)], axis=0)
         for w in W1s]
        + [jnp.zeros((HID - NE * D, HID), jnp.float32)], axis=0)
    b1sum = functools.reduce(jnp.add, b1s)[None, :]
    w2cat = jnp.concatenate(W2s, axis=0)
    b2sum = functools.reduce(jnp.add, b2s)[None, :]

    return _tc_kernel(agg_part, ct8, sel, v8, w1cat, b1sum, w2cat, b2sum)
